# Initial kernel scaffold; baseline (speedup 1.0000x reference)
import jax
import jax.numpy as jnp
from jax.experimental import pallas as pl


def kernel(original_pts, query_pts, W_in, b_in, W_b0, b_b0, W_b1, b_b1, W_b2, b_b2, R1, Rb1, R2, Rb2, R3, Rb3):
    raise NotImplementedError("write your pallas kernel here")


# speedup vs baseline: 32.5513x; 32.5513x over previous
import functools

import jax
import jax.numpy as jnp
from jax import lax
from jax.experimental import pallas as pl
from jax.experimental.pallas import tpu as pltpu
from jax.experimental.pallas import tpu_sc as plsc

B, N, M, C = 2, 4096, 4096, 128
K = 3
TM = 256
ROWS = B * M * K
NW = 32
RPW = ROWS // NW
CH = 128


def _feat_body(p_ref, win_ref, bin_ref, w0_ref, b0_ref, w1_ref, b1_ref,
               w2_ref, b2_ref, r1f_ref, r1g_ref, rb1_ref, z_ref, g_ref):
    p = p_ref[0]
    dn = (((0,), (0,)), ((), ()))
    f = jnp.maximum(
        lax.dot_general(win_ref[...], p, dn,
                        preferred_element_type=jnp.float32)
        + bin_ref[...].reshape(C, 1), 0.0)
    z = lax.dot_general(f, r1f_ref[0], dn,
                        preferred_element_type=jnp.float32)
    for i, (w_r, b_r) in enumerate(((w0_ref, b0_ref), (w1_ref, b1_ref),
                                    (w2_ref, b2_ref))):
        f = jnp.maximum(
            lax.dot_general(w_r[...], f, dn,
                            preferred_element_type=jnp.float32)
            + b_r[...].reshape(C, 1), 0.0)
        z = z + lax.dot_general(f, r1f_ref[i + 1], dn,
                                preferred_element_type=jnp.float32)
    z_ref[0] = z
    gmax = jnp.max(f, axis=1)
    g = lax.dot_general(gmax.reshape(1, C), r1g_ref[...],
                        (((1,), (0,)), ((), ())),
                        preferred_element_type=jnp.float32)
    g_ref[0] = g + rb1_ref[...]


def _feat_call(orig, w_in, b_in, w0, b0, w1, b1, w2, b2, r1f, r1g, rb1):
    return pl.pallas_call(
        _feat_body,
        grid=(B,),
        in_specs=[
            pl.BlockSpec((1, 3, N), lambda b: (b, 0, 0)),
            pl.BlockSpec((3, C), lambda b: (0, 0)),
            pl.BlockSpec((1, C), lambda b: (0, 0)),
            pl.BlockSpec((C, C), lambda b: (0, 0)),
            pl.BlockSpec((1, C), lambda b: (0, 0)),
            pl.BlockSpec((C, C), lambda b: (0, 0)),
            pl.BlockSpec((1, C), lambda b: (0, 0)),
            pl.BlockSpec((C, C), lambda b: (0, 0)),
            pl.BlockSpec((1, C), lambda b: (0, 0)),
            pl.BlockSpec((4, C, 256), lambda b: (0, 0, 0)),
            pl.BlockSpec((C, 256), lambda b: (0, 0)),
            pl.BlockSpec((1, 256), lambda b: (0, 0)),
        ],
        out_specs=[
            pl.BlockSpec((1, N, 256), lambda b: (b, 0, 0)),
            pl.BlockSpec((1, 1, 256), lambda b: (b, 0, 0)),
        ],
        out_shape=[
            jax.ShapeDtypeStruct((B, N, 256), jnp.float32),
            jax.ShapeDtypeStruct((B, 1, 256), jnp.float32),
        ],
    )(orig, w_in, b_in, w0, b0, w1, b1, w2, b2, r1f, r1g, rb1)


def _knn_body(q_ref, p_ref, idx_ref, w_ref):
    b = pl.program_id(0)
    q = q_ref[0]
    p = p_ref[0]
    q2 = jnp.sum(q * q, axis=0)
    p2 = jnp.sum(p * p, axis=0)
    inner = lax.dot_general(q, p, (((0,), (0,)), ((), ())),
                            preferred_element_type=jnp.float32)
    d2 = jnp.maximum(q2[:, None] + p2[None, :] - 2.0 * inner, 0.0)
    iota = lax.broadcasted_iota(jnp.int32, (TM, N), 1)
    recips = []
    for k in range(K):
        v = jnp.min(d2, axis=1)
        sel = d2 == v[:, None]
        i = jnp.min(jnp.where(sel, iota, N), axis=1)
        onehot = (iota == i[:, None]).astype(jnp.float32)
        if k < K - 1:
            d2 = jnp.where(iota == i[:, None], jnp.float32(3e38), d2)
        pk = lax.dot_general(onehot, p, (((1,), (1,)), ((), ())),
                             preferred_element_type=jnp.float32)
        dd0 = pk[:, 0] - q[0]
        dd1 = pk[:, 1] - q[1]
        dd2 = pk[:, 2] - q[2]
        dist = jnp.sqrt(dd0 * dd0 + dd1 * dd1 + dd2 * dd2 + 1e-12)
        recips.append(1.0 / (dist + 1e-8))
        idx_ref[0, k, :] = i + b * N
    rsum = recips[0] + recips[1] + recips[2]
    for k in range(K):
        w_ref[0, k, :] = recips[k] / rsum


def _knn_call(query, orig):
    return pl.pallas_call(
        _knn_body,
        grid=(B, M // TM),
        in_specs=[
            pl.BlockSpec((1, 3, TM), lambda b, t: (b, 0, t)),
            pl.BlockSpec((1, 3, N), lambda b, t: (b, 0, 0)),
        ],
        out_specs=[
            pl.BlockSpec((1, K, TM), lambda b, t: (b, 0, t)),
            pl.BlockSpec((1, K, TM), lambda b, t: (b, 0, t)),
        ],
        out_shape=[
            jax.ShapeDtypeStruct((B, K, M), jnp.int32),
            jax.ShapeDtypeStruct((B, K, M), jnp.float32),
        ],
    )(query, orig)


def _sc_gather(zflat, idxflat):
    mesh = plsc.VectorSubcoreMesh(core_axis_name="c", subcore_axis_name="s")

    @functools.partial(
        pl.kernel,
        out_type=jax.ShapeDtypeStruct((ROWS, 256), jnp.float32),
        mesh=mesh,
        scratch_types=[
            pltpu.VMEM((CH,), jnp.int32),
            pltpu.VMEM((CH, 256), jnp.float32),
            pltpu.SemaphoreType.DMA,
        ],
    )
    def gk(z_hbm, idx_hbm, out_hbm, idx_v, rows_v, sem):
        wid = lax.axis_index("s") * 2 + lax.axis_index("c")
        base = wid * RPW
        for i in range(RPW // CH):
            off = base + i * CH
            pltpu.sync_copy(idx_hbm.at[pl.ds(off, CH)], idx_v)
            pltpu.async_copy(z_hbm.at[idx_v], rows_v, sem).wait()
            pltpu.sync_copy(rows_v, out_hbm.at[pl.ds(off, CH)])

    return gk(zflat, idxflat)


def _reg_body(g3_ref, w_ref, q_ref, gcon_ref, r1q_ref, r2_ref, rb2_ref,
              r3_ref, rb3_ref, out_ref):
    acc = (w_ref[0, 0, :][:, None] * g3_ref[0, 0]
           + w_ref[0, 1, :][:, None] * g3_ref[0, 1]
           + w_ref[0, 2, :][:, None] * g3_ref[0, 2])
    acc = acc + lax.dot_general(q_ref[0], r1q_ref[...],
                                (((0,), (0,)), ((), ())),
                                preferred_element_type=jnp.float32)
    h1 = jnp.maximum(acc + gcon_ref[0], 0.0)
    h2 = jnp.maximum(
        lax.dot_general(h1, r2_ref[...], (((1,), (0,)), ((), ())),
                        preferred_element_type=jnp.float32)
        + rb2_ref[...], 0.0)
    o = lax.dot_general(h2, r3_ref[...], (((1,), (0,)), ((), ())),
                        preferred_element_type=jnp.float32)
    out_ref[0, 0, :] = o[:, 0] + rb3_ref[0, 0]


def _reg_call(g4, w, query, gcon, r1q, r2, rb2, r3, rb3):
    return pl.pallas_call(
        _reg_body,
        grid=(B, M // TM),
        in_specs=[
            pl.BlockSpec((1, K, TM, 256), lambda b, t: (b, 0, t, 0)),
            pl.BlockSpec((1, K, TM), lambda b, t: (b, 0, t)),
            pl.BlockSpec((1, 3, TM), lambda b, t: (b, 0, t)),
            pl.BlockSpec((1, 1, 256), lambda b, t: (b, 0, 0)),
            pl.BlockSpec((3, 256), lambda b, t: (0, 0)),
            pl.BlockSpec((256, 64), lambda b, t: (0, 0)),
            pl.BlockSpec((1, 64), lambda b, t: (0, 0)),
            pl.BlockSpec((64, 1), lambda b, t: (0, 0)),
            pl.BlockSpec((1, 1), lambda b, t: (0, 0)),
        ],
        out_specs=pl.BlockSpec((1, 1, TM), lambda b, t: (b, 0, t)),
        out_shape=jax.ShapeDtypeStruct((B, 1, M), jnp.float32),
    )(g4, w, query, gcon, r1q, r2, rb2, r3, rb3)


def kernel(original_pts, query_pts, W_in, b_in, W_b0, b_b0, W_b1, b_b1,
           W_b2, b_b2, R1, Rb1, R2, Rb2, R3, Rb3):
    r1q = R1[0:3]
    r1f = R1[3:3 + 4 * C].reshape(4, C, 256)
    r1g = R1[3 + 4 * C:]
    z, gcon = _feat_call(original_pts, W_in, b_in.reshape(1, C),
                         W_b0, b_b0.reshape(1, C), W_b1, b_b1.reshape(1, C),
                         W_b2, b_b2.reshape(1, C), r1f, r1g,
                         Rb1.reshape(1, 256))
    idx, w = _knn_call(query_pts, original_pts)
    g = _sc_gather(z.reshape(B * N, 256), idx.reshape(ROWS))
    out = _reg_call(g.reshape(B, K, M, 256), w, query_pts, gcon, r1q,
                    R2, Rb2.reshape(1, 64), R3, Rb3.reshape(1, 1))
    return out


# speedup vs baseline: 35.1783x; 1.0807x over previous
import functools

import jax
import jax.numpy as jnp
from jax import lax
from jax.experimental import pallas as pl
from jax.experimental.pallas import tpu as pltpu
from jax.experimental.pallas import tpu_sc as plsc

B, N, M, C = 2, 4096, 4096, 128
K = 3
TM = 512
ROWS = B * M * K
NW = 32
RPW = ROWS // NW
CH = 128


def _feat_body(p_ref, win_ref, bin_ref, w0_ref, b0_ref, w1_ref, b1_ref,
               w2_ref, b2_ref, r1f_ref, r1g_ref, rb1_ref, z_ref, g_ref):
    p = p_ref[0]
    dn = (((0,), (0,)), ((), ()))
    f = jnp.maximum(
        lax.dot_general(win_ref[...], p, dn,
                        preferred_element_type=jnp.float32)
        + bin_ref[...].reshape(C, 1), 0.0)
    z = lax.dot_general(f, r1f_ref[0], dn,
                        preferred_element_type=jnp.float32)
    for i, (w_r, b_r) in enumerate(((w0_ref, b0_ref), (w1_ref, b1_ref),
                                    (w2_ref, b2_ref))):
        f = jnp.maximum(
            lax.dot_general(w_r[...], f, dn,
                            preferred_element_type=jnp.float32)
            + b_r[...].reshape(C, 1), 0.0)
        z = z + lax.dot_general(f, r1f_ref[i + 1], dn,
                                preferred_element_type=jnp.float32)
    z_ref[0] = z
    gmax = jnp.max(f, axis=1)
    g = lax.dot_general(gmax.reshape(1, C), r1g_ref[...],
                        (((1,), (0,)), ((), ())),
                        preferred_element_type=jnp.float32)
    g_ref[0] = g + rb1_ref[...]


def _feat_call(orig, w_in, b_in, w0, b0, w1, b1, w2, b2, r1f, r1g, rb1):
    return pl.pallas_call(
        _feat_body,
        grid=(B,),
        in_specs=[
            pl.BlockSpec((1, 3, N), lambda b: (b, 0, 0)),
            pl.BlockSpec((3, C), lambda b: (0, 0)),
            pl.BlockSpec((1, C), lambda b: (0, 0)),
            pl.BlockSpec((C, C), lambda b: (0, 0)),
            pl.BlockSpec((1, C), lambda b: (0, 0)),
            pl.BlockSpec((C, C), lambda b: (0, 0)),
            pl.BlockSpec((1, C), lambda b: (0, 0)),
            pl.BlockSpec((C, C), lambda b: (0, 0)),
            pl.BlockSpec((1, C), lambda b: (0, 0)),
            pl.BlockSpec((4, C, 256), lambda b: (0, 0, 0)),
            pl.BlockSpec((C, 256), lambda b: (0, 0)),
            pl.BlockSpec((1, 256), lambda b: (0, 0)),
        ],
        out_specs=[
            pl.BlockSpec((1, N, 256), lambda b: (b, 0, 0)),
            pl.BlockSpec((1, 1, 256), lambda b: (b, 0, 0)),
        ],
        out_shape=[
            jax.ShapeDtypeStruct((B, N, 256), jnp.float32),
            jax.ShapeDtypeStruct((B, 1, 256), jnp.float32),
        ],
    )(orig, w_in, b_in, w0, b0, w1, b1, w2, b2, r1f, r1g, rb1)


def _knn_body(q_ref, p_ref, idx_ref, w_ref):
    b = pl.program_id(0)
    q = q_ref[0]
    p = p_ref[0]
    q2 = jnp.sum(q * q, axis=0)
    p2 = jnp.sum(p * p, axis=0)
    inner = lax.dot_general(q, p, (((0,), (0,)), ((), ())),
                            preferred_element_type=jnp.float32)
    d2 = jnp.maximum(q2[:, None] + p2[None, :] - 2.0 * inner, 0.0)
    iota = lax.broadcasted_iota(jnp.int32, (TM, N), 1)
    recips = []
    for k in range(K):
        v = jnp.min(d2, axis=1)
        sel = d2 == v[:, None]
        i = jnp.min(jnp.where(sel, iota, N), axis=1)
        m = iota == i[:, None]
        onehot = m.astype(jnp.float32)
        if k < K - 1:
            d2 = jnp.where(m, jnp.float32(3e38), d2)
        pk = lax.dot_general(onehot, p, (((1,), (1,)), ((), ())),
                             preferred_element_type=jnp.float32)
        dd0 = pk[:, 0] - q[0]
        dd1 = pk[:, 1] - q[1]
        dd2 = pk[:, 2] - q[2]
        dist = jnp.sqrt(dd0 * dd0 + dd1 * dd1 + dd2 * dd2 + 1e-12)
        recips.append(1.0 / (dist + 1e-8))
        idx_ref[0, k, :] = i + b * N
    rsum = recips[0] + recips[1] + recips[2]
    for k in range(K):
        w_ref[0, k, :] = recips[k] / rsum


def _knn_call(query, orig):
    return pl.pallas_call(
        _knn_body,
        grid=(B, M // TM),
        in_specs=[
            pl.BlockSpec((1, 3, TM), lambda b, t: (b, 0, t)),
            pl.BlockSpec((1, 3, N), lambda b, t: (b, 0, 0)),
        ],
        out_specs=[
            pl.BlockSpec((1, K, TM), lambda b, t: (b, 0, t)),
            pl.BlockSpec((1, K, TM), lambda b, t: (b, 0, t)),
        ],
        out_shape=[
            jax.ShapeDtypeStruct((B, K, M), jnp.int32),
            jax.ShapeDtypeStruct((B, K, M), jnp.float32),
        ],
    )(query, orig)


def _sc_gather(zflat, idxflat):
    mesh = plsc.VectorSubcoreMesh(core_axis_name="c", subcore_axis_name="s")

    @functools.partial(
        pl.kernel,
        out_type=jax.ShapeDtypeStruct((ROWS, 256), jnp.float32),
        mesh=mesh,
        scratch_types=[
            pltpu.VMEM((CH,), jnp.int32),
            pltpu.VMEM((CH, 256), jnp.float32),
            pltpu.SemaphoreType.DMA,
        ],
    )
    def gk(z_hbm, idx_hbm, gout_hbm, idx_v, rows_v, sem):
        wid = lax.axis_index("s") * 2 + lax.axis_index("c")
        base = wid * RPW
        for i in range(RPW // CH):
            off = base + i * CH
            pltpu.sync_copy(idx_hbm.at[pl.ds(off, CH)], idx_v)
            pltpu.async_copy(z_hbm.at[idx_v], rows_v, sem).wait()
            pltpu.sync_copy(rows_v, gout_hbm.at[pl.ds(off, CH)])

    return gk(zflat, idxflat)


def _reg_body(g3_ref, w_ref, q_ref, gcon_ref, r1q_ref, r2_ref, rb2_ref,
              r3_ref, rb3_ref, out_ref):
    q = q_ref[0]
    acc = (w_ref[0, 0, :][:, None] * g3_ref[0, 0]
           + w_ref[0, 1, :][:, None] * g3_ref[0, 1]
           + w_ref[0, 2, :][:, None] * g3_ref[0, 2])
    acc = acc + lax.dot_general(q, r1q_ref[...],
                                (((0,), (0,)), ((), ())),
                                preferred_element_type=jnp.float32)
    h1 = jnp.maximum(acc + gcon_ref[0], 0.0)
    h2 = jnp.maximum(
        lax.dot_general(h1, r2_ref[...], (((1,), (0,)), ((), ())),
                        preferred_element_type=jnp.float32)
        + rb2_ref[...], 0.0)
    o = lax.dot_general(h2, r3_ref[...], (((1,), (0,)), ((), ())),
                        preferred_element_type=jnp.float32)
    out_ref[0, 0, :] = o[:, 0] + rb3_ref[0, 0]


def _reg_call(g4, w, query, gcon, r1q, r2, rb2, r3, rb3):
    return pl.pallas_call(
        _reg_body,
        grid=(B, M // TM),
        in_specs=[
            pl.BlockSpec((1, K, TM, 256), lambda b, t: (b, 0, t, 0)),
            pl.BlockSpec((1, K, TM), lambda b, t: (b, 0, t)),
            pl.BlockSpec((1, 3, TM), lambda b, t: (b, 0, t)),
            pl.BlockSpec((1, 1, 256), lambda b, t: (b, 0, 0)),
            pl.BlockSpec((3, 256), lambda b, t: (0, 0)),
            pl.BlockSpec((256, 64), lambda b, t: (0, 0)),
            pl.BlockSpec((1, 64), lambda b, t: (0, 0)),
            pl.BlockSpec((64, 1), lambda b, t: (0, 0)),
            pl.BlockSpec((1, 1), lambda b, t: (0, 0)),
        ],
        out_specs=pl.BlockSpec((1, 1, TM), lambda b, t: (b, 0, t)),
        out_shape=jax.ShapeDtypeStruct((B, 1, M), jnp.float32),
    )(g4, w, query, gcon, r1q, r2, rb2, r3, rb3)


def kernel(original_pts, query_pts, W_in, b_in, W_b0, b_b0, W_b1, b_b1,
           W_b2, b_b2, R1, Rb1, R2, Rb2, R3, Rb3):
    r1q = R1[0:3]
    r1f = R1[3:3 + 4 * C].reshape(4, C, 256)
    r1g = R1[3 + 4 * C:]
    z, gcon = _feat_call(original_pts, W_in, b_in.reshape(1, C),
                         W_b0, b_b0.reshape(1, C), W_b1, b_b1.reshape(1, C),
                         W_b2, b_b2.reshape(1, C), r1f, r1g,
                         Rb1.reshape(1, 256))
    idx, w = _knn_call(query_pts, original_pts)
    g = _sc_gather(z.reshape(B * N, 256), idx.reshape(ROWS))
    out = _reg_call(g.reshape(B, K, M, 256), w, query_pts, gcon, r1q,
                    R2, Rb2.reshape(1, 64), R3, Rb3.reshape(1, 1))
    return out


# speedup vs baseline: 43.7079x; 1.2425x over previous
import functools

import jax
import jax.numpy as jnp
from jax import lax
from jax.experimental import pallas as pl
from jax.experimental.pallas import tpu as pltpu
from jax.experimental.pallas import tpu_sc as plsc

B, N, M, C = 2, 4096, 4096, 128
K = 3
TM = 512
ROWS = B * M * K
NW = 32
RPW = ROWS // NW
CH = 128


def _feat_body(p_ref, win_ref, bin_ref, w0_ref, b0_ref, w1_ref, b1_ref,
               w2_ref, b2_ref, r1f_ref, r1g_ref, rb1_ref, z_ref, g_ref):
    p = p_ref[0]
    dn = (((0,), (0,)), ((), ()))
    f = jnp.maximum(
        lax.dot_general(win_ref[...], p, dn,
                        preferred_element_type=jnp.float32)
        + bin_ref[...].reshape(C, 1), 0.0)
    z = lax.dot_general(f, r1f_ref[0], dn,
                        preferred_element_type=jnp.float32)
    for i, (w_r, b_r) in enumerate(((w0_ref, b0_ref), (w1_ref, b1_ref),
                                    (w2_ref, b2_ref))):
        f = jnp.maximum(
            lax.dot_general(w_r[...], f, dn,
                            preferred_element_type=jnp.float32)
            + b_r[...].reshape(C, 1), 0.0)
        z = z + lax.dot_general(f, r1f_ref[i + 1], dn,
                                preferred_element_type=jnp.float32)
    z_ref[0] = z
    gmax = jnp.max(f, axis=1)
    g = lax.dot_general(gmax.reshape(1, C), r1g_ref[...],
                        (((1,), (0,)), ((), ())),
                        preferred_element_type=jnp.float32)
    g_ref[0] = g + rb1_ref[...]


def _feat_call(orig, w_in, b_in, w0, b0, w1, b1, w2, b2, r1f, r1g, rb1):
    return pl.pallas_call(
        _feat_body,
        grid=(B,),
        in_specs=[
            pl.BlockSpec((1, 3, N), lambda b: (b, 0, 0)),
            pl.BlockSpec((3, C), lambda b: (0, 0)),
            pl.BlockSpec((1, C), lambda b: (0, 0)),
            pl.BlockSpec((C, C), lambda b: (0, 0)),
            pl.BlockSpec((1, C), lambda b: (0, 0)),
            pl.BlockSpec((C, C), lambda b: (0, 0)),
            pl.BlockSpec((1, C), lambda b: (0, 0)),
            pl.BlockSpec((C, C), lambda b: (0, 0)),
            pl.BlockSpec((1, C), lambda b: (0, 0)),
            pl.BlockSpec((4, C, 256), lambda b: (0, 0, 0)),
            pl.BlockSpec((C, 256), lambda b: (0, 0)),
            pl.BlockSpec((1, 256), lambda b: (0, 0)),
        ],
        out_specs=[
            pl.BlockSpec((1, N, 256), lambda b: (b, 0, 0)),
            pl.BlockSpec((1, 1, 256), lambda b: (b, 0, 0)),
        ],
        out_shape=[
            jax.ShapeDtypeStruct((B, N, 256), jnp.float32),
            jax.ShapeDtypeStruct((B, 1, 256), jnp.float32),
        ],
    )(orig, w_in, b_in, w0, b0, w1, b1, w2, b2, r1f, r1g, rb1)


def _knn_body(q_ref, p_ref, idx_ref):
    b = pl.program_id(0)
    q = q_ref[0]
    p = p_ref[0]
    q2 = jnp.sum(q * q, axis=0)
    p2 = jnp.sum(p * p, axis=0)
    inner = lax.dot_general(q, p, (((0,), (0,)), ((), ())),
                            preferred_element_type=jnp.float32)
    d2 = jnp.maximum(q2[:, None] + p2[None, :] - 2.0 * inner, 0.0)
    iota = lax.broadcasted_iota(jnp.int32, (TM, N), 1)
    for k in range(K):
        v = jnp.min(d2, axis=1)
        sel = d2 == v[:, None]
        i = jnp.min(jnp.where(sel, iota, N), axis=1)
        idx_ref[0, k, :] = i + b * N
        if k < K - 1:
            d2 = jnp.where(sel, jnp.float32(3e38), d2)


def _knn_call(query, orig):
    return pl.pallas_call(
        _knn_body,
        grid=(B, M // TM),
        in_specs=[
            pl.BlockSpec((1, 3, TM), lambda b, t: (b, 0, t)),
            pl.BlockSpec((1, 3, N), lambda b, t: (b, 0, 0)),
        ],
        out_specs=pl.BlockSpec((1, K, TM), lambda b, t: (b, 0, t)),
        out_shape=jax.ShapeDtypeStruct((B, K, M), jnp.int32),
    )(query, orig)


def _sc_gather(zflat, ct, idxflat):
    mesh = plsc.VectorSubcoreMesh(core_axis_name="c", subcore_axis_name="s")

    @functools.partial(
        pl.kernel,
        out_type=[
            jax.ShapeDtypeStruct((ROWS, 256), jnp.float32),
            jax.ShapeDtypeStruct((ROWS, 128), jnp.float32),
        ],
        mesh=mesh,
        scratch_types=[
            pltpu.VMEM((CH,), jnp.int32),
            pltpu.VMEM((CH, 256), jnp.float32),
            pltpu.VMEM((CH, 128), jnp.float32),
            pltpu.SemaphoreType.DMA,
        ],
    )
    def gk(z_hbm, ct_hbm, idx_hbm, gout_hbm, xout_hbm, idx_v, rows_v, xyz_v,
           sem):
        wid = lax.axis_index("s") * 2 + lax.axis_index("c")
        base = wid * RPW
        for i in range(RPW // CH):
            off = base + i * CH
            pltpu.sync_copy(idx_hbm.at[pl.ds(off, CH)], idx_v)
            cp1 = pltpu.async_copy(z_hbm.at[idx_v], rows_v, sem)
            cp2 = pltpu.async_copy(ct_hbm.at[idx_v], xyz_v, sem)
            cp1.wait()
            cp2.wait()
            pltpu.sync_copy(rows_v, gout_hbm.at[pl.ds(off, CH)])
            pltpu.sync_copy(xyz_v, xout_hbm.at[pl.ds(off, CH)])

    return gk(zflat, ct, idxflat)


def _reg_body(g3_ref, xyz_ref, q_ref, gcon_ref, r1q_ref, r2_ref, rb2_ref,
              r3_ref, rb3_ref, out_ref):
    q = q_ref[0]
    recips = []
    for k in range(K):
        xk = xyz_ref[0, k]
        dd0 = xk[:, 0] - q[0]
        dd1 = xk[:, 1] - q[1]
        dd2 = xk[:, 2] - q[2]
        dist = jnp.sqrt(dd0 * dd0 + dd1 * dd1 + dd2 * dd2 + 1e-12)
        recips.append(1.0 / (dist + 1e-8))
    rsum = recips[0] + recips[1] + recips[2]
    acc = ((recips[0] / rsum)[:, None] * g3_ref[0, 0]
           + (recips[1] / rsum)[:, None] * g3_ref[0, 1]
           + (recips[2] / rsum)[:, None] * g3_ref[0, 2])
    acc = acc + lax.dot_general(q, r1q_ref[...],
                                (((0,), (0,)), ((), ())),
                                preferred_element_type=jnp.float32)
    h1 = jnp.maximum(acc + gcon_ref[0], 0.0)
    h2 = jnp.maximum(
        lax.dot_general(h1, r2_ref[...], (((1,), (0,)), ((), ())),
                        preferred_element_type=jnp.float32)
        + rb2_ref[...], 0.0)
    o = lax.dot_general(h2, r3_ref[...], (((1,), (0,)), ((), ())),
                        preferred_element_type=jnp.float32)
    out_ref[0, 0, :] = o[:, 0] + rb3_ref[0, 0]


def _reg_call(g4, xyz, query, gcon, r1q, r2, rb2, r3, rb3):
    return pl.pallas_call(
        _reg_body,
        grid=(B, M // TM),
        in_specs=[
            pl.BlockSpec((1, K, TM, 256), lambda b, t: (b, 0, t, 0)),
            pl.BlockSpec((1, K, TM, 128), lambda b, t: (b, 0, t, 0)),
            pl.BlockSpec((1, 3, TM), lambda b, t: (b, 0, t)),
            pl.BlockSpec((1, 1, 256), lambda b, t: (b, 0, 0)),
            pl.BlockSpec((3, 256), lambda b, t: (0, 0)),
            pl.BlockSpec((256, 64), lambda b, t: (0, 0)),
            pl.BlockSpec((1, 64), lambda b, t: (0, 0)),
            pl.BlockSpec((64, 1), lambda b, t: (0, 0)),
            pl.BlockSpec((1, 1), lambda b, t: (0, 0)),
        ],
        out_specs=pl.BlockSpec((1, 1, TM), lambda b, t: (b, 0, t)),
        out_shape=jax.ShapeDtypeStruct((B, 1, M), jnp.float32),
    )(g4, xyz, query, gcon, r1q, r2, rb2, r3, rb3)


def kernel(original_pts, query_pts, W_in, b_in, W_b0, b_b0, W_b1, b_b1,
           W_b2, b_b2, R1, Rb1, R2, Rb2, R3, Rb3):
    r1q = R1[0:3]
    r1f = R1[3:3 + 4 * C].reshape(4, C, 256)
    r1g = R1[3 + 4 * C:]
    z, gcon = _feat_call(original_pts, W_in, b_in.reshape(1, C),
                         W_b0, b_b0.reshape(1, C), W_b1, b_b1.reshape(1, C),
                         W_b2, b_b2.reshape(1, C), r1f, r1g,
                         Rb1.reshape(1, 256))
    idx = _knn_call(query_pts, original_pts)
    ct = jnp.pad(jnp.transpose(original_pts, (0, 2, 1)),
                 ((0, 0), (0, 0), (0, 125))).reshape(B * N, 128)
    g, xyz = _sc_gather(z.reshape(B * N, 256), ct, idx.reshape(ROWS))
    out = _reg_call(g.reshape(B, K, M, 256), xyz.reshape(B, K, M, 128),
                    query_pts, gcon, r1q, R2, Rb2.reshape(1, 64), R3,
                    Rb3.reshape(1, 1))
    return out


# speedup vs baseline: 44.4922x; 1.0179x over previous
import functools

import jax
import jax.numpy as jnp
from jax import lax
from jax.experimental import pallas as pl
from jax.experimental.pallas import tpu as pltpu
from jax.experimental.pallas import tpu_sc as plsc

B, N, M, C = 2, 4096, 4096, 128
K = 3
TM = 512
ROWS = B * M * K
NW = 32
RPW = ROWS // NW
CH = 128


def _feat_body(p_ref, win_ref, bin_ref, w0_ref, b0_ref, w1_ref, b1_ref,
               w2_ref, b2_ref, r1f_ref, r1g_ref, rb1_ref, z_ref, g_ref):
    p = p_ref[0]
    dn = (((0,), (0,)), ((), ()))
    f = jnp.maximum(
        lax.dot_general(win_ref[...], p, dn,
                        preferred_element_type=jnp.float32)
        + bin_ref[...].reshape(C, 1), 0.0)
    z = lax.dot_general(f, r1f_ref[0], dn,
                        preferred_element_type=jnp.float32)
    for i, (w_r, b_r) in enumerate(((w0_ref, b0_ref), (w1_ref, b1_ref),
                                    (w2_ref, b2_ref))):
        f = jnp.maximum(
            lax.dot_general(w_r[...], f, dn,
                            preferred_element_type=jnp.float32)
            + b_r[...].reshape(C, 1), 0.0)
        z = z + lax.dot_general(f, r1f_ref[i + 1], dn,
                                preferred_element_type=jnp.float32)
    z_ref[0] = z
    gmax = jnp.max(f, axis=1)
    g = lax.dot_general(gmax.reshape(1, C), r1g_ref[...],
                        (((1,), (0,)), ((), ())),
                        preferred_element_type=jnp.float32)
    g_ref[0] = g + rb1_ref[...]


def _feat_call(orig, w_in, b_in, w0, b0, w1, b1, w2, b2, r1f, r1g, rb1):
    return pl.pallas_call(
        _feat_body,
        grid=(B,),
        in_specs=[
            pl.BlockSpec((1, 3, N), lambda b: (b, 0, 0)),
            pl.BlockSpec((3, C), lambda b: (0, 0)),
            pl.BlockSpec((1, C), lambda b: (0, 0)),
            pl.BlockSpec((C, C), lambda b: (0, 0)),
            pl.BlockSpec((1, C), lambda b: (0, 0)),
            pl.BlockSpec((C, C), lambda b: (0, 0)),
            pl.BlockSpec((1, C), lambda b: (0, 0)),
            pl.BlockSpec((C, C), lambda b: (0, 0)),
            pl.BlockSpec((1, C), lambda b: (0, 0)),
            pl.BlockSpec((4, C, 256), lambda b: (0, 0, 0)),
            pl.BlockSpec((C, 256), lambda b: (0, 0)),
            pl.BlockSpec((1, 256), lambda b: (0, 0)),
        ],
        out_specs=[
            pl.BlockSpec((1, N, 256), lambda b: (b, 0, 0)),
            pl.BlockSpec((1, 1, 256), lambda b: (b, 0, 0)),
        ],
        out_shape=[
            jax.ShapeDtypeStruct((B, N, 256), jnp.float32),
            jax.ShapeDtypeStruct((B, 1, 256), jnp.float32),
        ],
    )(orig, w_in, b_in, w0, b0, w1, b1, w2, b2, r1f, r1g, rb1)


def _knn_body(q_ref, p_ref, idx_ref):
    b = pl.program_id(0)
    q = q_ref[0]
    p = p_ref[0]
    q2 = jnp.sum(q * q, axis=0)
    p2 = jnp.sum(p * p, axis=0)
    inner = lax.dot_general(q, p, (((0,), (0,)), ((), ())),
                            preferred_element_type=jnp.float32)
    d2 = jnp.maximum(q2[:, None] + p2[None, :] - 2.0 * inner, 0.0)
    iota = lax.broadcasted_iota(jnp.int32, (TM, N), 1)
    for k in range(K):
        v = jnp.min(d2, axis=1)
        sel = d2 == v[:, None]
        i = jnp.min(jnp.where(sel, iota, N), axis=1)
        idx_ref[0, k, :] = i + b * N
        if k < K - 1:
            d2 = jnp.where(sel, jnp.float32(3e38), d2)


def _knn_call(query, orig):
    return pl.pallas_call(
        _knn_body,
        grid=(B, M // TM),
        in_specs=[
            pl.BlockSpec((1, 3, TM), lambda b, t: (b, 0, t)),
            pl.BlockSpec((1, 3, N), lambda b, t: (b, 0, 0)),
        ],
        out_specs=pl.BlockSpec((1, K, TM), lambda b, t: (b, 0, t)),
        out_shape=jax.ShapeDtypeStruct((B, K, M), jnp.int32),
    )(query, orig)


def _sc_gather(zflat, ct, idxflat):
    mesh = plsc.VectorSubcoreMesh(core_axis_name="c", subcore_axis_name="s")

    @functools.partial(
        pl.kernel,
        out_type=[
            jax.ShapeDtypeStruct((ROWS, 256), jnp.float32),
            jax.ShapeDtypeStruct((ROWS, 128), jnp.float32),
        ],
        mesh=mesh,
        scratch_types=[
            pltpu.VMEM((2, CH), jnp.int32),
            pltpu.VMEM((2, CH, 256), jnp.float32),
            pltpu.VMEM((2, CH, 128), jnp.float32),
            pltpu.SemaphoreType.DMA,
            pltpu.SemaphoreType.DMA,
        ],
    )
    def gk(z_hbm, ct_hbm, idx_hbm, gout_hbm, xout_hbm, idx_v, rows_v, xyz_v,
           sem0, sem1):
        wid = lax.axis_index("s") * 2 + lax.axis_index("c")
        base = wid * RPW
        nch = RPW // CH
        sems = (sem0, sem1)

        def fire(i):
            s = i % 2
            off = base + i * CH
            pltpu.sync_copy(idx_hbm.at[pl.ds(off, CH)], idx_v.at[s])
            pltpu.async_copy(z_hbm.at[idx_v.at[s]], rows_v.at[s], sems[s])
            pltpu.async_copy(ct_hbm.at[idx_v.at[s]], xyz_v.at[s], sems[s])

        def drain(i):
            s = i % 2
            off = base + i * CH
            pltpu.make_async_copy(z_hbm.at[idx_v.at[s]], rows_v.at[s],
                                  sems[s]).wait()
            pltpu.make_async_copy(ct_hbm.at[idx_v.at[s]], xyz_v.at[s],
                                  sems[s]).wait()
            pltpu.sync_copy(rows_v.at[s], gout_hbm.at[pl.ds(off, CH)])
            pltpu.sync_copy(xyz_v.at[s], xout_hbm.at[pl.ds(off, CH)])

        fire(0)
        for i in range(1, nch):
            fire(i)
            drain(i - 1)
        drain(nch - 1)

    return gk(zflat, ct, idxflat)


def _reg_body(g3_ref, xyz_ref, q_ref, gcon_ref, r1q_ref, r2_ref, rb2_ref,
              r3_ref, rb3_ref, out_ref):
    q = q_ref[0]
    recips = []
    for k in range(K):
        xk = xyz_ref[0, k]
        dd0 = xk[:, 0] - q[0]
        dd1 = xk[:, 1] - q[1]
        dd2 = xk[:, 2] - q[2]
        dist = jnp.sqrt(dd0 * dd0 + dd1 * dd1 + dd2 * dd2 + 1e-12)
        recips.append(1.0 / (dist + 1e-8))
    rsum = recips[0] + recips[1] + recips[2]
    acc = ((recips[0] / rsum)[:, None] * g3_ref[0, 0]
           + (recips[1] / rsum)[:, None] * g3_ref[0, 1]
           + (recips[2] / rsum)[:, None] * g3_ref[0, 2])
    acc = acc + lax.dot_general(q, r1q_ref[...],
                                (((0,), (0,)), ((), ())),
                                preferred_element_type=jnp.float32)
    h1 = jnp.maximum(acc + gcon_ref[0], 0.0)
    h2 = jnp.maximum(
        lax.dot_general(h1, r2_ref[...], (((1,), (0,)), ((), ())),
                        preferred_element_type=jnp.float32)
        + rb2_ref[...], 0.0)
    o = lax.dot_general(h2, r3_ref[...], (((1,), (0,)), ((), ())),
                        preferred_element_type=jnp.float32)
    out_ref[0, 0, :] = o[:, 0] + rb3_ref[0, 0]


def _reg_call(g4, xyz, query, gcon, r1q, r2, rb2, r3, rb3):
    return pl.pallas_call(
        _reg_body,
        grid=(B, M // TM),
        in_specs=[
            pl.BlockSpec((1, K, TM, 256), lambda b, t: (b, 0, t, 0)),
            pl.BlockSpec((1, K, TM, 128), lambda b, t: (b, 0, t, 0)),
            pl.BlockSpec((1, 3, TM), lambda b, t: (b, 0, t)),
            pl.BlockSpec((1, 1, 256), lambda b, t: (b, 0, 0)),
            pl.BlockSpec((3, 256), lambda b, t: (0, 0)),
            pl.BlockSpec((256, 64), lambda b, t: (0, 0)),
            pl.BlockSpec((1, 64), lambda b, t: (0, 0)),
            pl.BlockSpec((64, 1), lambda b, t: (0, 0)),
            pl.BlockSpec((1, 1), lambda b, t: (0, 0)),
        ],
        out_specs=pl.BlockSpec((1, 1, TM), lambda b, t: (b, 0, t)),
        out_shape=jax.ShapeDtypeStruct((B, 1, M), jnp.float32),
    )(g4, xyz, query, gcon, r1q, r2, rb2, r3, rb3)


def kernel(original_pts, query_pts, W_in, b_in, W_b0, b_b0, W_b1, b_b1,
           W_b2, b_b2, R1, Rb1, R2, Rb2, R3, Rb3):
    r1q = R1[0:3]
    r1f = R1[3:3 + 4 * C].reshape(4, C, 256)
    r1g = R1[3 + 4 * C:]
    z, gcon = _feat_call(original_pts, W_in, b_in.reshape(1, C),
                         W_b0, b_b0.reshape(1, C), W_b1, b_b1.reshape(1, C),
                         W_b2, b_b2.reshape(1, C), r1f, r1g,
                         Rb1.reshape(1, 256))
    idx = _knn_call(query_pts, original_pts)
    ct = jnp.pad(jnp.transpose(original_pts, (0, 2, 1)),
                 ((0, 0), (0, 0), (0, 125))).reshape(B * N, 128)
    g, xyz = _sc_gather(z.reshape(B * N, 256), ct, idx.reshape(ROWS))
    out = _reg_call(g.reshape(B, K, M, 256), xyz.reshape(B, K, M, 128),
                    query_pts, gcon, r1q, R2, Rb2.reshape(1, 64), R3,
                    Rb3.reshape(1, 1))
    return out


# speedup vs baseline: 44.8848x; 1.0088x over previous
import functools

import jax
import jax.numpy as jnp
from jax import lax
from jax.experimental import pallas as pl
from jax.experimental.pallas import tpu as pltpu
from jax.experimental.pallas import tpu_sc as plsc

B, N, M, C = 2, 4096, 4096, 128
K = 3
TM = 512
ROWS = B * M * K
NW = 32
RPW = ROWS // NW
CH = 128


def _feat_body(p_ref, win_ref, bin_ref, w0_ref, b0_ref, w1_ref, b1_ref,
               w2_ref, b2_ref, r1f_ref, r1g_ref, rb1_ref, z_ref, g_ref):
    p = p_ref[0]
    dn = (((0,), (0,)), ((), ()))
    f = jnp.maximum(
        lax.dot_general(win_ref[...], p, dn,
                        preferred_element_type=jnp.float32)
        + bin_ref[...].reshape(C, 1), 0.0)
    z = lax.dot_general(f, r1f_ref[0], dn,
                        preferred_element_type=jnp.float32)
    for i, (w_r, b_r) in enumerate(((w0_ref, b0_ref), (w1_ref, b1_ref),
                                    (w2_ref, b2_ref))):
        f = jnp.maximum(
            lax.dot_general(w_r[...], f, dn,
                            preferred_element_type=jnp.float32)
            + b_r[...].reshape(C, 1), 0.0)
        z = z + lax.dot_general(f, r1f_ref[i + 1], dn,
                                preferred_element_type=jnp.float32)
    z_ref[0] = z
    gmax = jnp.max(f, axis=1)
    g = lax.dot_general(gmax.reshape(1, C), r1g_ref[...],
                        (((1,), (0,)), ((), ())),
                        preferred_element_type=jnp.float32)
    g_ref[0] = g + rb1_ref[...]


def _feat_call(orig, w_in, b_in, w0, b0, w1, b1, w2, b2, r1f, r1g, rb1):
    return pl.pallas_call(
        _feat_body,
        grid=(B,),
        in_specs=[
            pl.BlockSpec((1, 3, N), lambda b: (b, 0, 0)),
            pl.BlockSpec((3, C), lambda b: (0, 0)),
            pl.BlockSpec((1, C), lambda b: (0, 0)),
            pl.BlockSpec((C, C), lambda b: (0, 0)),
            pl.BlockSpec((1, C), lambda b: (0, 0)),
            pl.BlockSpec((C, C), lambda b: (0, 0)),
            pl.BlockSpec((1, C), lambda b: (0, 0)),
            pl.BlockSpec((C, C), lambda b: (0, 0)),
            pl.BlockSpec((1, C), lambda b: (0, 0)),
            pl.BlockSpec((4, C, 256), lambda b: (0, 0, 0)),
            pl.BlockSpec((C, 256), lambda b: (0, 0)),
            pl.BlockSpec((1, 256), lambda b: (0, 0)),
        ],
        out_specs=[
            pl.BlockSpec((1, N, 256), lambda b: (b, 0, 0)),
            pl.BlockSpec((1, 1, 256), lambda b: (b, 0, 0)),
        ],
        out_shape=[
            jax.ShapeDtypeStruct((B, N, 256), jnp.float32),
            jax.ShapeDtypeStruct((B, 1, 256), jnp.float32),
        ],
    )(orig, w_in, b_in, w0, b0, w1, b1, w2, b2, r1f, r1g, rb1)


def _knn_body(q_ref, p_ref, idx_ref):
    b = pl.program_id(0)
    q = q_ref[0]
    p = p_ref[0]
    p2 = jnp.sum(p * p, axis=0)
    inner = lax.dot_general(-2.0 * q, p, (((0,), (0,)), ((), ())),
                            preferred_element_type=jnp.float32)
    s = inner + p2[None, :]
    iota = lax.broadcasted_iota(jnp.int32, (TM, N), 1)
    for k in range(K):
        v = jnp.min(s, axis=1)
        sel = s == v[:, None]
        i = jnp.min(jnp.where(sel, iota, N), axis=1)
        idx_ref[0, k, :] = i + b * N
        if k < K - 1:
            s = jnp.where(sel, jnp.float32(3e38), s)


def _knn_call(query, orig):
    return pl.pallas_call(
        _knn_body,
        grid=(B, M // TM),
        in_specs=[
            pl.BlockSpec((1, 3, TM), lambda b, t: (b, 0, t)),
            pl.BlockSpec((1, 3, N), lambda b, t: (b, 0, 0)),
        ],
        out_specs=pl.BlockSpec((1, K, TM), lambda b, t: (b, 0, t)),
        out_shape=jax.ShapeDtypeStruct((B, K, M), jnp.int32),
    )(query, orig)


def _sc_gather(zflat, ct, idxflat):
    mesh = plsc.VectorSubcoreMesh(core_axis_name="c", subcore_axis_name="s")

    @functools.partial(
        pl.kernel,
        out_type=[
            jax.ShapeDtypeStruct((ROWS, 256), jnp.float32),
            jax.ShapeDtypeStruct((ROWS, 128), jnp.float32),
        ],
        mesh=mesh,
        scratch_types=[
            pltpu.VMEM((2, CH), jnp.int32),
            pltpu.VMEM((2, CH, 256), jnp.float32),
            pltpu.VMEM((2, CH, 128), jnp.float32),
            pltpu.SemaphoreType.DMA,
            pltpu.SemaphoreType.DMA,
        ],
    )
    def gk(z_hbm, ct_hbm, idx_hbm, gout_hbm, xout_hbm, idx_v, rows_v, xyz_v,
           sem0, sem1):
        wid = lax.axis_index("s") * 2 + lax.axis_index("c")
        base = wid * RPW
        nch = RPW // CH
        sems = (sem0, sem1)

        def fire(i):
            s = i % 2
            off = base + i * CH
            pltpu.sync_copy(idx_hbm.at[pl.ds(off, CH)], idx_v.at[s])
            pltpu.async_copy(z_hbm.at[idx_v.at[s]], rows_v.at[s], sems[s])
            pltpu.async_copy(ct_hbm.at[idx_v.at[s]], xyz_v.at[s], sems[s])

        def drain(i):
            s = i % 2
            off = base + i * CH
            pltpu.make_async_copy(z_hbm.at[idx_v.at[s]], rows_v.at[s],
                                  sems[s]).wait()
            pltpu.make_async_copy(ct_hbm.at[idx_v.at[s]], xyz_v.at[s],
                                  sems[s]).wait()
            pltpu.sync_copy(rows_v.at[s], gout_hbm.at[pl.ds(off, CH)])
            pltpu.sync_copy(xyz_v.at[s], xout_hbm.at[pl.ds(off, CH)])

        fire(0)
        for i in range(1, nch):
            fire(i)
            drain(i - 1)
        drain(nch - 1)

    return gk(zflat, ct, idxflat)


def _reg_body(g3_ref, xyz_ref, q_ref, gcon_ref, r1q_ref, r2_ref, rb2_ref,
              r3_ref, rb3_ref, out_ref):
    q = q_ref[0]
    recips = []
    for k in range(K):
        xk = xyz_ref[0, k]
        dd0 = xk[:, 0] - q[0]
        dd1 = xk[:, 1] - q[1]
        dd2 = xk[:, 2] - q[2]
        dist = jnp.sqrt(dd0 * dd0 + dd1 * dd1 + dd2 * dd2 + 1e-12)
        recips.append(1.0 / (dist + 1e-8))
    rsum = recips[0] + recips[1] + recips[2]
    acc = ((recips[0] / rsum)[:, None] * g3_ref[0, 0]
           + (recips[1] / rsum)[:, None] * g3_ref[0, 1]
           + (recips[2] / rsum)[:, None] * g3_ref[0, 2])
    acc = acc + lax.dot_general(q, r1q_ref[...],
                                (((0,), (0,)), ((), ())),
                                preferred_element_type=jnp.float32)
    h1 = jnp.maximum(acc + gcon_ref[0], 0.0)
    h2 = jnp.maximum(
        lax.dot_general(h1, r2_ref[...], (((1,), (0,)), ((), ())),
                        preferred_element_type=jnp.float32)
        + rb2_ref[...], 0.0)
    o = lax.dot_general(h2, r3_ref[...], (((1,), (0,)), ((), ())),
                        preferred_element_type=jnp.float32)
    out_ref[0, 0, :] = o[:, 0] + rb3_ref[0, 0]


def _reg_call(g4, xyz, query, gcon, r1q, r2, rb2, r3, rb3):
    return pl.pallas_call(
        _reg_body,
        grid=(B, M // TM),
        in_specs=[
            pl.BlockSpec((1, K, TM, 256), lambda b, t: (b, 0, t, 0)),
            pl.BlockSpec((1, K, TM, 128), lambda b, t: (b, 0, t, 0)),
            pl.BlockSpec((1, 3, TM), lambda b, t: (b, 0, t)),
            pl.BlockSpec((1, 1, 256), lambda b, t: (b, 0, 0)),
            pl.BlockSpec((3, 256), lambda b, t: (0, 0)),
            pl.BlockSpec((256, 64), lambda b, t: (0, 0)),
            pl.BlockSpec((1, 64), lambda b, t: (0, 0)),
            pl.BlockSpec((64, 1), lambda b, t: (0, 0)),
            pl.BlockSpec((1, 1), lambda b, t: (0, 0)),
        ],
        out_specs=pl.BlockSpec((1, 1, TM), lambda b, t: (b, 0, t)),
        out_shape=jax.ShapeDtypeStruct((B, 1, M), jnp.float32),
    )(g4, xyz, query, gcon, r1q, r2, rb2, r3, rb3)


def kernel(original_pts, query_pts, W_in, b_in, W_b0, b_b0, W_b1, b_b1,
           W_b2, b_b2, R1, Rb1, R2, Rb2, R3, Rb3):
    r1q = R1[0:3]
    r1f = R1[3:3 + 4 * C].reshape(4, C, 256)
    r1g = R1[3 + 4 * C:]
    z, gcon = _feat_call(original_pts, W_in, b_in.reshape(1, C),
                         W_b0, b_b0.reshape(1, C), W_b1, b_b1.reshape(1, C),
                         W_b2, b_b2.reshape(1, C), r1f, r1g,
                         Rb1.reshape(1, 256))
    idx = _knn_call(query_pts, original_pts)
    ct = jnp.pad(jnp.transpose(original_pts, (0, 2, 1)),
                 ((0, 0), (0, 0), (0, 125))).reshape(B * N, 128)
    g, xyz = _sc_gather(z.reshape(B * N, 256), ct, idx.reshape(ROWS))
    out = _reg_call(g.reshape(B, K, M, 256), xyz.reshape(B, K, M, 128),
                    query_pts, gcon, r1q, R2, Rb2.reshape(1, 64), R3,
                    Rb3.reshape(1, 1))
    return out


# speedup vs baseline: 51.0828x; 1.1381x over previous
import functools

import jax
import jax.numpy as jnp
from jax import lax
from jax.experimental import pallas as pl
from jax.experimental.pallas import tpu as pltpu
from jax.experimental.pallas import tpu_sc as plsc

B, N, M, C = 2, 4096, 4096, 128
K = 3
TM = 512
ROWS = B * M * K
NW = 32
RPW = ROWS // NW
CH = 128


def _feat_body(p_ref, win_ref, bin_ref, w0_ref, b0_ref, w1_ref, b1_ref,
               w2_ref, b2_ref, r1f_ref, r1g_ref, rb1_ref, z_ref, g_ref):
    p = p_ref[0]
    dn = (((0,), (0,)), ((), ()))
    f = jnp.maximum(
        lax.dot_general(win_ref[...], p, dn,
                        preferred_element_type=jnp.float32)
        + bin_ref[...].reshape(C, 1), 0.0)
    z = lax.dot_general(f, r1f_ref[0], dn,
                        preferred_element_type=jnp.float32)
    for i, (w_r, b_r) in enumerate(((w0_ref, b0_ref), (w1_ref, b1_ref),
                                    (w2_ref, b2_ref))):
        f = jnp.maximum(
            lax.dot_general(w_r[...], f, dn,
                            preferred_element_type=jnp.float32)
            + b_r[...].reshape(C, 1), 0.0)
        z = z + lax.dot_general(f, r1f_ref[i + 1], dn,
                                preferred_element_type=jnp.float32)
    z_ref[0] = z
    gmax = jnp.max(f, axis=1)
    g = lax.dot_general(gmax.reshape(1, C), r1g_ref[...],
                        (((1,), (0,)), ((), ())),
                        preferred_element_type=jnp.float32)
    g_ref[0] = g + rb1_ref[...]


def _feat_call(orig, w_in, b_in, w0, b0, w1, b1, w2, b2, r1f, r1g, rb1):
    return pl.pallas_call(
        _feat_body,
        grid=(B,),
        in_specs=[
            pl.BlockSpec((1, 3, N), lambda b: (b, 0, 0)),
            pl.BlockSpec((3, C), lambda b: (0, 0)),
            pl.BlockSpec((1, C), lambda b: (0, 0)),
            pl.BlockSpec((C, C), lambda b: (0, 0)),
            pl.BlockSpec((1, C), lambda b: (0, 0)),
            pl.BlockSpec((C, C), lambda b: (0, 0)),
            pl.BlockSpec((1, C), lambda b: (0, 0)),
            pl.BlockSpec((C, C), lambda b: (0, 0)),
            pl.BlockSpec((1, C), lambda b: (0, 0)),
            pl.BlockSpec((4, C, 256), lambda b: (0, 0, 0)),
            pl.BlockSpec((C, 256), lambda b: (0, 0)),
            pl.BlockSpec((1, 256), lambda b: (0, 0)),
        ],
        out_specs=[
            pl.BlockSpec((1, N, 256), lambda b: (b, 0, 0)),
            pl.BlockSpec((1, 1, 256), lambda b: (b, 0, 0)),
        ],
        out_shape=[
            jax.ShapeDtypeStruct((B, N, 256), jnp.float32),
            jax.ShapeDtypeStruct((B, 1, 256), jnp.float32),
        ],
    )(orig, w_in, b_in, w0, b0, w1, b1, w2, b2, r1f, r1g, rb1)


def _knn_body(q_ref, p_ref, idx_ref):
    b = pl.program_id(0)
    q = q_ref[0]
    p = p_ref[0]
    p2 = jnp.sum(p * p, axis=0)
    inner = lax.dot_general(-2.0 * q, p, (((0,), (0,)), ((), ())),
                            preferred_element_type=jnp.float32)
    s = inner + p2[None, :]
    big = jnp.float32(3e38)
    s0 = jnp.full((TM, 128), big, jnp.float32)
    s1 = jnp.full((TM, 128), big, jnp.float32)
    j0 = jnp.zeros((TM, 128), jnp.int32)
    j1 = jnp.zeros((TM, 128), jnp.int32)
    for j in range(N // 128):
        v = s[:, j * 128:(j + 1) * 128]
        c0 = v < s0
        c1 = v < s1
        s1 = jnp.where(c0, s0, jnp.where(c1, v, s1))
        j1 = jnp.where(c0, j0, jnp.where(c1, j, j1))
        s0 = jnp.where(c0, v, s0)
        j0 = jnp.where(c0, j, j0)
    lane = lax.broadcasted_iota(jnp.int32, (TM, 128), 1)
    vals = jnp.concatenate([s0, s1], axis=1)
    gidx = jnp.concatenate([j0 * 128 + lane, j1 * 128 + lane], axis=1)
    for k in range(K):
        v = jnp.min(vals, axis=1)
        sel = vals == v[:, None]
        i = jnp.min(jnp.where(sel, gidx, jnp.int32(1 << 30)), axis=1)
        idx_ref[0, k, :] = i + b * N
        if k < K - 1:
            vals = jnp.where(sel, big, vals)


def _knn_call(query, orig):
    return pl.pallas_call(
        _knn_body,
        grid=(B, M // TM),
        in_specs=[
            pl.BlockSpec((1, 3, TM), lambda b, t: (b, 0, t)),
            pl.BlockSpec((1, 3, N), lambda b, t: (b, 0, 0)),
        ],
        out_specs=pl.BlockSpec((1, K, TM), lambda b, t: (b, 0, t)),
        out_shape=jax.ShapeDtypeStruct((B, K, M), jnp.int32),
    )(query, orig)


def _sc_gather(zflat, ct, idxflat):
    mesh = plsc.VectorSubcoreMesh(core_axis_name="c", subcore_axis_name="s")

    @functools.partial(
        pl.kernel,
        out_type=[
            jax.ShapeDtypeStruct((ROWS, 256), jnp.float32),
            jax.ShapeDtypeStruct((ROWS, 128), jnp.float32),
        ],
        mesh=mesh,
        scratch_types=[
            pltpu.VMEM((2, CH), jnp.int32),
            pltpu.VMEM((2, CH, 256), jnp.float32),
            pltpu.VMEM((2, CH, 128), jnp.float32),
            pltpu.SemaphoreType.DMA,
            pltpu.SemaphoreType.DMA,
        ],
    )
    def gk(z_hbm, ct_hbm, idx_hbm, gout_hbm, xout_hbm, idx_v, rows_v, xyz_v,
           sem0, sem1):
        wid = lax.axis_index("s") * 2 + lax.axis_index("c")
        base = wid * RPW
        nch = RPW // CH
        sems = (sem0, sem1)

        def fire(i):
            s = i % 2
            off = base + i * CH
            pltpu.sync_copy(idx_hbm.at[pl.ds(off, CH)], idx_v.at[s])
            pltpu.async_copy(z_hbm.at[idx_v.at[s]], rows_v.at[s], sems[s])
            pltpu.async_copy(ct_hbm.at[idx_v.at[s]], xyz_v.at[s], sems[s])

        def drain(i):
            s = i % 2
            off = base + i * CH
            pltpu.make_async_copy(z_hbm.at[idx_v.at[s]], rows_v.at[s],
                                  sems[s]).wait()
            pltpu.make_async_copy(ct_hbm.at[idx_v.at[s]], xyz_v.at[s],
                                  sems[s]).wait()
            pltpu.sync_copy(rows_v.at[s], gout_hbm.at[pl.ds(off, CH)])
            pltpu.sync_copy(xyz_v.at[s], xout_hbm.at[pl.ds(off, CH)])

        fire(0)
        for i in range(1, nch):
            fire(i)
            drain(i - 1)
        drain(nch - 1)

    return gk(zflat, ct, idxflat)


def _reg_body(g3_ref, xyz_ref, q_ref, gcon_ref, r1q_ref, r2_ref, rb2_ref,
              r3_ref, rb3_ref, out_ref):
    q = q_ref[0]
    recips = []
    for k in range(K):
        xk = xyz_ref[0, k]
        dd0 = xk[:, 0] - q[0]
        dd1 = xk[:, 1] - q[1]
        dd2 = xk[:, 2] - q[2]
        dist = jnp.sqrt(dd0 * dd0 + dd1 * dd1 + dd2 * dd2 + 1e-12)
        recips.append(1.0 / (dist + 1e-8))
    rsum = recips[0] + recips[1] + recips[2]
    acc = ((recips[0] / rsum)[:, None] * g3_ref[0, 0]
           + (recips[1] / rsum)[:, None] * g3_ref[0, 1]
           + (recips[2] / rsum)[:, None] * g3_ref[0, 2])
    acc = acc + lax.dot_general(q, r1q_ref[...],
                                (((0,), (0,)), ((), ())),
                                preferred_element_type=jnp.float32)
    h1 = jnp.maximum(acc + gcon_ref[0], 0.0)
    h2 = jnp.maximum(
        lax.dot_general(h1, r2_ref[...], (((1,), (0,)), ((), ())),
                        preferred_element_type=jnp.float32)
        + rb2_ref[...], 0.0)
    o = lax.dot_general(h2, r3_ref[...], (((1,), (0,)), ((), ())),
                        preferred_element_type=jnp.float32)
    out_ref[0, 0, :] = o[:, 0] + rb3_ref[0, 0]


def _reg_call(g4, xyz, query, gcon, r1q, r2, rb2, r3, rb3):
    return pl.pallas_call(
        _reg_body,
        grid=(B, M // TM),
        in_specs=[
            pl.BlockSpec((1, K, TM, 256), lambda b, t: (b, 0, t, 0)),
            pl.BlockSpec((1, K, TM, 128), lambda b, t: (b, 0, t, 0)),
            pl.BlockSpec((1, 3, TM), lambda b, t: (b, 0, t)),
            pl.BlockSpec((1, 1, 256), lambda b, t: (b, 0, 0)),
            pl.BlockSpec((3, 256), lambda b, t: (0, 0)),
            pl.BlockSpec((256, 64), lambda b, t: (0, 0)),
            pl.BlockSpec((1, 64), lambda b, t: (0, 0)),
            pl.BlockSpec((64, 1), lambda b, t: (0, 0)),
            pl.BlockSpec((1, 1), lambda b, t: (0, 0)),
        ],
        out_specs=pl.BlockSpec((1, 1, TM), lambda b, t: (b, 0, t)),
        out_shape=jax.ShapeDtypeStruct((B, 1, M), jnp.float32),
    )(g4, xyz, query, gcon, r1q, r2, rb2, r3, rb3)


def kernel(original_pts, query_pts, W_in, b_in, W_b0, b_b0, W_b1, b_b1,
           W_b2, b_b2, R1, Rb1, R2, Rb2, R3, Rb3):
    r1q = R1[0:3]
    r1f = R1[3:3 + 4 * C].reshape(4, C, 256)
    r1g = R1[3 + 4 * C:]
    z, gcon = _feat_call(original_pts, W_in, b_in.reshape(1, C),
                         W_b0, b_b0.reshape(1, C), W_b1, b_b1.reshape(1, C),
                         W_b2, b_b2.reshape(1, C), r1f, r1g,
                         Rb1.reshape(1, 256))
    idx = _knn_call(query_pts, original_pts)
    ct = jnp.pad(jnp.transpose(original_pts, (0, 2, 1)),
                 ((0, 0), (0, 0), (0, 125))).reshape(B * N, 128)
    g, xyz = _sc_gather(z.reshape(B * N, 256), ct, idx.reshape(ROWS))
    out = _reg_call(g.reshape(B, K, M, 256), xyz.reshape(B, K, M, 128),
                    query_pts, gcon, r1q, R2, Rb2.reshape(1, 64), R3,
                    Rb3.reshape(1, 1))
    return out
